# baseline (device time: 71901 ns/iter reference)
import jax
import jax.numpy as jnp
from jax import lax
from jax.experimental import pallas as pl
from jax.experimental.pallas import tpu as pltpu

N_DEV = 16
DH = 64

_NEXT = [4, 2, 6, 0, 8, 1, 10, 3, 12, 5, 14, 7, 13, 9, 15, 11]
_PREV = [3, 5, 1, 7, 0, 9, 2, 11, 4, 13, 6, 15, 8, 12, 10, 14]


def kernel(x, Wq, Wo, K_ext, V_ext):
    B, S, D = x.shape
    H = Wq.shape[1] // DH
    Skv = K_ext.shape[1]
    my = lax.axis_index("i")

    k = lax.dynamic_slice_in_dim(K_ext, my * H, H, axis=2)
    k = jnp.transpose(k, (0, 2, 1, 3)).reshape(B * H, Skv, DH).astype(jnp.bfloat16)
    v = lax.dynamic_slice_in_dim(V_ext, my * H, H, axis=2)
    v = jnp.transpose(v, (0, 2, 1, 3)).reshape(B * H, Skv, DH).astype(jnp.bfloat16)

    xb = x.astype(jnp.bfloat16)
    wq = (Wq * 0.125).astype(jnp.bfloat16)
    wo = Wo.astype(jnp.bfloat16)

    nxt = jnp.asarray(_NEXT, dtype=jnp.int32)[my].reshape(1)
    prv = jnp.asarray(_PREV, dtype=jnp.int32)[my].reshape(1)

    def body(x_ref, wq_ref, wo_ref, k_ref, v_ref, nxt_ref, prv_ref, out_ref,
             xbuf, abuf, xsend, xrecv, asend, arecv):
        right = nxt_ref[0]
        left = prv_ref[0]

        barrier = pltpu.get_barrier_semaphore()
        pl.semaphore_signal(barrier, inc=1, device_id=(left,),
                            device_id_type=pl.DeviceIdType.MESH)
        pl.semaphore_signal(barrier, inc=1, device_id=(right,),
                            device_id_type=pl.DeviceIdType.MESH)
        pl.semaphore_wait(barrier, 2)

        def partial_for(xblk):
            x2 = xblk.reshape(B * S, D)
            q = jnp.dot(x2, wq_ref[...], preferred_element_type=jnp.float32)
            q = q.astype(jnp.bfloat16).reshape(B, S, H, DH)
            q = jnp.transpose(q, (0, 2, 1, 3)).reshape(B * H, S, DH)
            s = lax.dot_general(
                q, k_ref[...],
                dimension_numbers=(((2,), (2,)), ((0,), (0,))),
                preferred_element_type=jnp.float32,
            )
            p = jnp.exp(s)
            l = jnp.sum(p, axis=-1, keepdims=True)
            o = lax.dot_general(
                p.astype(jnp.bfloat16), v_ref[...],
                dimension_numbers=(((2,), (1,)), ((0,), (0,))),
                preferred_element_type=jnp.float32,
            )
            o = (o * (1.0 / l)).astype(jnp.bfloat16).reshape(B, H, S, DH)
            o = jnp.transpose(o, (0, 2, 1, 3)).reshape(B * S, H * DH)
            r = jnp.dot(o, wo_ref[...], preferred_element_type=jnp.float32)
            return r.reshape(B, S, D)

        def make_x(h):
            xr = pltpu.make_async_remote_copy(
                src_ref=xbuf.at[h, 0], dst_ref=xbuf.at[h + 1, 0],
                send_sem=xsend.at[h, 0], recv_sem=xrecv.at[h, 0],
                device_id=(right,), device_id_type=pl.DeviceIdType.MESH,
            )
            xl = pltpu.make_async_remote_copy(
                src_ref=xbuf.at[h, 1], dst_ref=xbuf.at[h + 1, 1],
                send_sem=xsend.at[h, 1], recv_sem=xrecv.at[h, 1],
                device_id=(left,), device_id_type=pl.DeviceIdType.MESH,
            )
            return xr, xl

        D2 = D // 2

        def make_a(h, c):
            dst = h + 1 if h < N_DEV - 1 else 0
            ar = pltpu.make_async_remote_copy(
                src_ref=abuf.at[h, 0, :, pl.ds(c * D2, D2)],
                dst_ref=abuf.at[dst, 0, :, pl.ds(c * D2, D2)],
                send_sem=asend.at[h, 0, c], recv_sem=arecv.at[h, 0, c],
                device_id=(right,), device_id_type=pl.DeviceIdType.MESH,
            )
            al = pltpu.make_async_remote_copy(
                src_ref=abuf.at[h, 1, :, pl.ds(c * D2, D2)],
                dst_ref=abuf.at[dst, 1, :, pl.ds(c * D2, D2)],
                send_sem=asend.at[h, 1, c], recv_sem=arecv.at[h, 1, c],
                device_id=(left,), device_id_type=pl.DeviceIdType.MESH,
            )
            return ar, al

        xbuf[0] = x_ref[...]
        xd = make_x(0)
        xd[0].start()
        xd[1].start()
        out_ref[...] = partial_for(x_ref[...])

        apend = None
        for h in range(N_DEV):
            if h <= N_DEV - 2:
                xd[0].wait()
                xd[1].wait()
                if h <= N_DEV - 3:
                    xd = make_x(h + 1)
                    xd[0].start()
                    xd[1].start()
                part = partial_for(xbuf[h + 1]).astype(jnp.bfloat16)
            nxt_pend = []
            for c in range(2):
                lo, hi = c * D2, (c + 1) * D2
                if apend is not None:
                    apend[c][0].wait()
                    apend[c][1].wait()
                if h <= N_DEV - 2:
                    if h == 0:
                        abuf[1, :, :, lo:hi] = part[:, :, lo:hi]
                    else:
                        abuf[h + 1, :, :, lo:hi] = (
                            abuf[h + 1, :, :, lo:hi] + part[:, :, lo:hi])
                    ad = make_a(h + 1, c)
                    ad[0].start()
                    ad[1].start()
                    nxt_pend.append(ad)
            apend = nxt_pend or None

        out_ref[...] = out_ref[...] + abuf[0].astype(jnp.float32)

    return pl.pallas_call(
        body,
        out_shape=jax.ShapeDtypeStruct((B, S, D), jnp.float32),
        in_specs=[pl.BlockSpec(memory_space=pltpu.VMEM)] * 5
        + [pl.BlockSpec(memory_space=pltpu.SMEM)] * 2,
        out_specs=pl.BlockSpec(memory_space=pltpu.VMEM),
        scratch_shapes=[
            pltpu.VMEM((N_DEV, B, S, D), jnp.bfloat16),
            pltpu.VMEM((N_DEV, B, S, D), jnp.bfloat16),
            pltpu.SemaphoreType.DMA((N_DEV, 2)),
            pltpu.SemaphoreType.DMA((N_DEV, 2)),
            pltpu.SemaphoreType.DMA((N_DEV, 2, 2)),
            pltpu.SemaphoreType.DMA((N_DEV, 2, 2)),
        ],
        compiler_params=pltpu.CompilerParams(collective_id=0),
    )(xb, wq, wo, k, v, nxt, prv)


# device time: 71848 ns/iter; 1.0007x vs baseline; 1.0007x over previous
import jax
import jax.numpy as jnp
from jax import lax
from jax.experimental import pallas as pl
from jax.experimental.pallas import tpu as pltpu

N_DEV = 16
DH = 64

_NEXT = [4, 2, 6, 0, 8, 1, 10, 3, 12, 5, 14, 7, 13, 9, 15, 11]
_PREV = [3, 5, 1, 7, 0, 9, 2, 11, 4, 13, 6, 15, 8, 12, 10, 14]


def kernel(x, Wq, Wo, K_ext, V_ext):
    B, S, D = x.shape
    H = Wq.shape[1] // DH
    Skv = K_ext.shape[1]
    my = lax.axis_index("i")

    k = lax.dynamic_slice_in_dim(K_ext, my * H, H, axis=2)
    k = jnp.transpose(k, (0, 2, 1, 3)).reshape(B * H, Skv, DH).astype(jnp.bfloat16)
    v = lax.dynamic_slice_in_dim(V_ext, my * H, H, axis=2)
    v = jnp.transpose(v, (0, 2, 1, 3)).reshape(B * H, Skv, DH).astype(jnp.bfloat16)

    xb = x.astype(jnp.bfloat16)
    wq = (Wq * 0.125).astype(jnp.bfloat16)
    wo = Wo.astype(jnp.bfloat16)

    nxt = jnp.asarray(_NEXT, dtype=jnp.int32)[my].reshape(1)
    prv = jnp.asarray(_PREV, dtype=jnp.int32)[my].reshape(1)

    def body(x_ref, wq_ref, wo_ref, k_ref, v_ref, nxt_ref, prv_ref, out_ref,
             xbuf, abuf, xsend, xrecv, asend, arecv):
        right = nxt_ref[0]
        left = prv_ref[0]

        barrier = pltpu.get_barrier_semaphore()
        pl.semaphore_signal(barrier, inc=1, device_id=(left,),
                            device_id_type=pl.DeviceIdType.MESH)
        pl.semaphore_signal(barrier, inc=1, device_id=(right,),
                            device_id_type=pl.DeviceIdType.MESH)
        pl.semaphore_wait(barrier, 2)

        def partial_for(xblk):
            x2 = xblk.reshape(B * S, D)
            q = jnp.dot(x2, wq_ref[...], preferred_element_type=jnp.float32)
            q = q.astype(jnp.bfloat16).reshape(B, S, H, DH)
            q = jnp.transpose(q, (0, 2, 1, 3)).reshape(B * H, S, DH)
            s = lax.dot_general(
                q, k_ref[...],
                dimension_numbers=(((2,), (2,)), ((0,), (0,))),
                preferred_element_type=jnp.float32,
            )
            p = jnp.exp(s)
            l = jnp.sum(p, axis=-1, keepdims=True)
            o = lax.dot_general(
                p.astype(jnp.bfloat16), v_ref[...],
                dimension_numbers=(((2,), (1,)), ((0,), (0,))),
                preferred_element_type=jnp.float32,
            )
            o = (o * (1.0 / l)).astype(jnp.bfloat16).reshape(B, H, S, DH)
            o = jnp.transpose(o, (0, 2, 1, 3)).reshape(B * S, H * DH)
            r = jnp.dot(o, wo_ref[...], preferred_element_type=jnp.float32)
            return r.reshape(B, S, D)

        def make_x(h):
            xr = pltpu.make_async_remote_copy(
                src_ref=xbuf.at[h, 0], dst_ref=xbuf.at[h + 1, 0],
                send_sem=xsend.at[h, 0], recv_sem=xrecv.at[h, 0],
                device_id=(right,), device_id_type=pl.DeviceIdType.MESH,
            )
            xl = pltpu.make_async_remote_copy(
                src_ref=xbuf.at[h, 1], dst_ref=xbuf.at[h + 1, 1],
                send_sem=xsend.at[h, 1], recv_sem=xrecv.at[h, 1],
                device_id=(left,), device_id_type=pl.DeviceIdType.MESH,
            )
            return xr, xl

        def make_a(h):
            dst = h + 1 if h < N_DEV - 1 else 0
            ar = pltpu.make_async_remote_copy(
                src_ref=abuf.at[h, 0], dst_ref=abuf.at[dst, 0],
                send_sem=asend.at[h, 0], recv_sem=arecv.at[h, 0],
                device_id=(right,), device_id_type=pl.DeviceIdType.MESH,
            )
            al = pltpu.make_async_remote_copy(
                src_ref=abuf.at[h, 1], dst_ref=abuf.at[dst, 1],
                send_sem=asend.at[h, 1], recv_sem=arecv.at[h, 1],
                device_id=(left,), device_id_type=pl.DeviceIdType.MESH,
            )
            return ar, al

        xbuf[0] = x_ref[...]
        xd = make_x(0)
        xd[0].start()
        xd[1].start()
        out_ref[...] = partial_for(x_ref[...])

        for h in range(N_DEV):
            if h >= 1:
                ad = make_a(h)
                ad[0].start()
                ad[1].start()
            if h <= N_DEV - 2:
                xd[0].wait()
                xd[1].wait()
                if h <= N_DEV - 3:
                    xd = make_x(h + 1)
                    xd[0].start()
                    xd[1].start()
                part = partial_for(xbuf[h + 1]).astype(jnp.bfloat16)
            if h >= 1:
                ad[0].wait()
                ad[1].wait()
            if h == 0:
                abuf[1] = part
            elif h <= N_DEV - 2:
                abuf[h + 1] = abuf[h + 1] + part

        out_ref[...] = out_ref[...] + abuf[0].astype(jnp.float32)

    return pl.pallas_call(
        body,
        out_shape=jax.ShapeDtypeStruct((B, S, D), jnp.float32),
        in_specs=[pl.BlockSpec(memory_space=pltpu.VMEM)] * 5
        + [pl.BlockSpec(memory_space=pltpu.SMEM)] * 2,
        out_specs=pl.BlockSpec(memory_space=pltpu.VMEM),
        scratch_shapes=[
            pltpu.VMEM((N_DEV, B, S, D), jnp.bfloat16),
            pltpu.VMEM((N_DEV, B, S, D), jnp.bfloat16),
            pltpu.SemaphoreType.DMA((N_DEV, 2)),
            pltpu.SemaphoreType.DMA((N_DEV, 2)),
            pltpu.SemaphoreType.DMA((N_DEV, 2)),
            pltpu.SemaphoreType.DMA((N_DEV, 2)),
        ],
        compiler_params=pltpu.CompilerParams(collective_id=0),
    )(xb, wq, wo, k, v, nxt, prv)
